# baseline (speedup 1.0000x reference)
import functools

import jax
import jax.numpy as jnp
from jax import lax
from jax.experimental import pallas as pl
from jax.experimental.pallas import tpu as pltpu
from jax.experimental.pallas import tpu_sc as plsc

N_NODES = 10000
D = 128
E = 320000

NC = 2
NS = 16
NW = NC * NS

CHUNK = 128
CPT = 80
EPT = CPT * CHUNK
E_PAD = NW * EPT
ROWS_PER_TILE = 632
ACC_ROWS = NS * ROWS_PER_TILE
DEG_RPT = 640
DEG_ROWS = NS * DEG_RPT
TRASH = N_NODES

_mesh = plsc.VectorSubcoreMesh(core_axis_name="c", subcore_axis_name="s")


@functools.partial(
    pl.kernel,
    out_type=jax.ShapeDtypeStruct((NC, DEG_ROWS), jnp.float32),
    mesh=_mesh,
    scratch_types=[
        pltpu.VMEM((CHUNK,), jnp.float32),
        pltpu.VMEM((CPT, CHUNK), jnp.int32),
        pltpu.VMEM_SHARED((DEG_ROWS,), jnp.float32),
    ],
)
def _deg_sc(dst_hbm, zeros_hbm, out_hbm, ones_v, didx, acc):
    cid = lax.axis_index("c")
    sid = lax.axis_index("s")
    wid = sid * NC + cid
    for j in range(CHUNK // 16):
        ones_v[pl.ds(j * 16, 16)] = jnp.ones((16,), jnp.float32)
    pltpu.sync_copy(zeros_hbm, acc.at[pl.ds(sid * DEG_RPT, DEG_RPT)])
    pltpu.sync_copy(dst_hbm.at[wid], didx)
    plsc.subcore_barrier()

    def body(c, carry):
        pltpu.sync_copy(ones_v, acc.at[didx.at[c]], add=True)
        return carry

    lax.fori_loop(0, CPT, body, 0)
    plsc.subcore_barrier()
    pltpu.sync_copy(
        acc.at[pl.ds(sid * DEG_RPT, DEG_RPT)],
        out_hbm.at[cid, pl.ds(sid * DEG_RPT, DEG_RPT)],
    )


@functools.partial(
    pl.kernel,
    out_type=jax.ShapeDtypeStruct((NC, ACC_ROWS, D), jnp.float32),
    mesh=_mesh,
    scratch_types=[
        pltpu.VMEM((CPT // 2, CHUNK), jnp.int32),
        pltpu.VMEM((CPT, CHUNK), jnp.int32),
        pltpu.VMEM((CHUNK, D), jnp.float32),
        pltpu.VMEM((CHUNK, D), jnp.float32),
        pltpu.SemaphoreType.DMA,
        pltpu.SemaphoreType.DMA,
        pltpu.SemaphoreType.DMA,
        pltpu.SemaphoreType.DMA,
        pltpu.VMEM_SHARED((ACC_ROWS, D), jnp.float32),
    ],
)
def _agg_sc(y_hbm, src_hbm, dst_hbm, zeros_hbm, out_hbm,
            sidx, didx, rows_a, rows_b,
            sem_ga, sem_gb, sem_sa, sem_sb, acc):
    cid = lax.axis_index("c")
    sid = lax.axis_index("s")
    wid = sid * NC + cid
    pltpu.sync_copy(zeros_hbm, acc.at[pl.ds(sid * ROWS_PER_TILE, ROWS_PER_TILE)])
    pltpu.sync_copy(dst_hbm.at[wid], didx)
    plsc.subcore_barrier()

    for half in range(2):
        pltpu.sync_copy(src_hbm.at[wid, half], sidx)

        def body(u, carry):
            c = half * (CPT // 2) + 2 * u
            g0 = pltpu.async_copy(y_hbm.at[sidx.at[2 * u]], rows_a, sem_ga)
            g1 = pltpu.async_copy(y_hbm.at[sidx.at[2 * u + 1]], rows_b, sem_gb)
            g0.wait()
            s0 = pltpu.async_copy(rows_a, acc.at[didx.at[c]], sem_sa, add=True)
            g1.wait()
            s1 = pltpu.async_copy(rows_b, acc.at[didx.at[c + 1]], sem_sb,
                                  add=True)
            s0.wait()
            s1.wait()
            return carry

        lax.fori_loop(0, CPT // 4, body, 0)

    plsc.subcore_barrier()
    pltpu.sync_copy(
        acc.at[pl.ds(sid * ROWS_PER_TILE, ROWS_PER_TILE)],
        out_hbm.at[cid, pl.ds(sid * ROWS_PER_TILE, ROWS_PER_TILE)],
    )


def kernel(x, edge_index, W1, b1, W2, b2):
    src = edge_index[0].astype(jnp.int32)
    dst = edge_index[1].astype(jnp.int32)
    pad = E_PAD - E
    srcp = jnp.concatenate([src, jnp.zeros((pad,), jnp.int32)])
    dstp = jnp.concatenate([dst, jnp.full((pad,), TRASH, jnp.int32)])
    srcp = srcp.reshape(NW, 2, CPT // 2, CHUNK)
    dstp = dstp.reshape(NW, CPT, CHUNK)

    zeros1 = jnp.zeros((DEG_RPT,), jnp.float32)
    zeros2 = jnp.zeros((ROWS_PER_TILE, D), jnp.float32)

    deg_parts = _deg_sc(dstp, zeros1)
    deg = deg_parts[0, :N_NODES] + deg_parts[1, :N_NODES] + 1.0
    dis = lax.rsqrt(deg)[:, None]

    y1 = (x @ W1) * dis
    agg1 = _agg_sc(y1, srcp, dstp, zeros2)
    h = dis * (agg1[0, :N_NODES] + agg1[1, :N_NODES] + y1) + b1
    h = jnp.maximum(h, 0.0)

    y2 = (h @ W2) * dis
    agg2 = _agg_sc(y2, srcp, dstp, zeros2)
    return dis * (agg2[0, :N_NODES] + agg2[1, :N_NODES] + y2) + b2


# speedup vs baseline: 1.4269x; 1.4269x over previous
import functools

import jax
import jax.numpy as jnp
from jax import lax
from jax.experimental import pallas as pl
from jax.experimental.pallas import tpu as pltpu
from jax.experimental.pallas import tpu_sc as plsc

N_NODES = 10000
D = 128
E = 320000

NC = 2
NS = 16
NW = NC * NS

CHUNK = 128
CPT = 79
EPT = CPT * CHUNK
E_PAD = NW * EPT
ROWS_PER_TILE = 632
ACC_ROWS = NS * ROWS_PER_TILE
DEG_RPT = 640
DEG_ROWS = NS * DEG_RPT
TRASH = N_NODES

_mesh = plsc.VectorSubcoreMesh(core_axis_name="c", subcore_axis_name="s")


@functools.partial(
    pl.kernel,
    out_type=jax.ShapeDtypeStruct((NC, DEG_ROWS), jnp.float32),
    mesh=_mesh,
    scratch_types=[
        pltpu.VMEM((CHUNK,), jnp.float32),
        pltpu.VMEM((CPT, CHUNK), jnp.int32),
        pltpu.VMEM_SHARED((DEG_ROWS,), jnp.float32),
    ],
)
def _deg_sc(dst_hbm, zeros_hbm, out_hbm, ones_v, didx, acc):
    cid = lax.axis_index("c")
    sid = lax.axis_index("s")
    wid = sid * NC + cid
    for j in range(CHUNK // 16):
        ones_v[pl.ds(j * 16, 16)] = jnp.ones((16,), jnp.float32)
    pltpu.sync_copy(zeros_hbm, acc.at[pl.ds(sid * DEG_RPT, DEG_RPT)])
    pltpu.sync_copy(dst_hbm.at[wid], didx)
    plsc.subcore_barrier()

    def body(c, carry):
        pltpu.sync_copy(ones_v, acc.at[didx.at[c]], add=True)
        return carry

    lax.fori_loop(0, CPT, body, 0)
    plsc.subcore_barrier()
    pltpu.sync_copy(
        acc.at[pl.ds(sid * DEG_RPT, DEG_RPT)],
        out_hbm.at[cid, pl.ds(sid * DEG_RPT, DEG_RPT)],
    )


@functools.partial(
    pl.kernel,
    out_type=jax.ShapeDtypeStruct((NC, ACC_ROWS, D), jnp.float32),
    mesh=_mesh,
    scratch_types=[
        pltpu.VMEM((CPT, CHUNK), jnp.int32),
        pltpu.VMEM((CPT, CHUNK), jnp.int32),
        pltpu.VMEM((CHUNK, D), jnp.float32),
        pltpu.SemaphoreType.DMA,
        pltpu.VMEM_SHARED((ACC_ROWS, D), jnp.float32),
    ],
)
def _agg_sc(y_hbm, src_hbm, dst_hbm, zeros_hbm, out_hbm,
            sidx, didx, rows, sem, acc):
    cid = lax.axis_index("c")
    sid = lax.axis_index("s")
    wid = sid * NC + cid
    pltpu.sync_copy(zeros_hbm, acc.at[pl.ds(sid * ROWS_PER_TILE, ROWS_PER_TILE)])
    pltpu.sync_copy(src_hbm.at[wid], sidx)
    pltpu.sync_copy(dst_hbm.at[wid], didx)
    plsc.subcore_barrier()

    def body(c, carry):
        pltpu.async_copy(y_hbm.at[sidx.at[c]], rows, sem).wait()
        pltpu.sync_copy(rows, acc.at[didx.at[c]], add=True)
        return carry

    lax.fori_loop(0, CPT, body, 0)

    plsc.subcore_barrier()
    pltpu.sync_copy(
        acc.at[pl.ds(sid * ROWS_PER_TILE, ROWS_PER_TILE)],
        out_hbm.at[cid, pl.ds(sid * ROWS_PER_TILE, ROWS_PER_TILE)],
    )


def kernel(x, edge_index, W1, b1, W2, b2):
    src = edge_index[0].astype(jnp.int32)
    dst = edge_index[1].astype(jnp.int32)
    pad = E_PAD - E
    srcp = jnp.concatenate([src, jnp.zeros((pad,), jnp.int32)])
    dstp = jnp.concatenate([dst, jnp.full((pad,), TRASH, jnp.int32)])
    srcp = srcp.reshape(NW, CPT, CHUNK)
    dstp = dstp.reshape(NW, CPT, CHUNK)

    zeros1 = jnp.zeros((DEG_RPT,), jnp.float32)
    zeros2 = jnp.zeros((ROWS_PER_TILE, D), jnp.float32)

    deg_parts = _deg_sc(dstp, zeros1)
    deg = deg_parts[0, :N_NODES] + deg_parts[1, :N_NODES] + 1.0
    dis = lax.rsqrt(deg)[:, None]

    y1 = (x @ W1) * dis
    agg1 = _agg_sc(y1, srcp, dstp, zeros2)
    h = dis * (agg1[0, :N_NODES] + agg1[1, :N_NODES] + y1) + b1
    h = jnp.maximum(h, 0.0)

    y2 = (h @ W2) * dis
    agg2 = _agg_sc(y2, srcp, dstp, zeros2)
    return dis * (agg2[0, :N_NODES] + agg2[1, :N_NODES] + y2) + b2


# speedup vs baseline: 1.5073x; 1.0563x over previous
import functools

import jax
import jax.numpy as jnp
from jax import lax
from jax.experimental import pallas as pl
from jax.experimental.pallas import tpu as pltpu
from jax.experimental.pallas import tpu_sc as plsc

N_NODES = 10000
D = 128
E = 320000

NC = 2
NS = 16
NW = NC * NS

CHUNK = 128
CPT = 79
EPT = CPT * CHUNK
E_PAD = NW * EPT
ROWS_PER_TILE = 640
ACC_ROWS = NS * ROWS_PER_TILE
DEG_RPT = 640
DEG_ROWS = NS * DEG_RPT
TRASH = N_NODES

_mesh = plsc.VectorSubcoreMesh(core_axis_name="c", subcore_axis_name="s")


@functools.partial(
    pl.kernel,
    out_type=jax.ShapeDtypeStruct((NC, DEG_ROWS), jnp.float32),
    mesh=_mesh,
    scratch_types=[
        pltpu.VMEM((CHUNK,), jnp.float32),
        pltpu.VMEM((CPT, CHUNK), jnp.int32),
        pltpu.VMEM_SHARED((DEG_ROWS,), jnp.float32),
    ],
)
def _deg_sc(dst_hbm, zeros_hbm, out_hbm, ones_v, didx, acc):
    cid = lax.axis_index("c")
    sid = lax.axis_index("s")
    wid = sid * NC + cid
    for j in range(CHUNK // 16):
        ones_v[pl.ds(j * 16, 16)] = jnp.ones((16,), jnp.float32)
    pltpu.sync_copy(zeros_hbm, acc.at[pl.ds(sid * DEG_RPT, DEG_RPT)])
    pltpu.sync_copy(dst_hbm.at[wid], didx)
    plsc.subcore_barrier()

    def body(c, carry):
        pltpu.sync_copy(ones_v, acc.at[didx.at[c]], add=True)
        return carry

    lax.fori_loop(0, CPT, body, 0)
    plsc.subcore_barrier()
    pltpu.sync_copy(
        acc.at[pl.ds(sid * DEG_RPT, DEG_RPT)],
        out_hbm.at[cid, pl.ds(sid * DEG_RPT, DEG_RPT)],
    )


@functools.partial(
    pl.kernel,
    out_type=jax.ShapeDtypeStruct((NC, ACC_ROWS, D), jnp.float32),
    mesh=_mesh,
    scratch_types=[
        pltpu.VMEM((CPT, CHUNK), jnp.int32),
        pltpu.VMEM((CPT, CHUNK), jnp.int32),
        pltpu.VMEM((CHUNK, D), jnp.float32),
        pltpu.SemaphoreType.DMA,
        pltpu.VMEM_SHARED((ACC_ROWS, D), jnp.float32),
    ],
)
def _agg_sc(y_hbm, src_hbm, dst_hbm, zeros_hbm, out_hbm,
            sidx, didx, rows, sem, acc):
    cid = lax.axis_index("c")
    sid = lax.axis_index("s")
    wid = sid * NC + cid
    pltpu.sync_copy(zeros_hbm, acc.at[pl.ds(sid * ROWS_PER_TILE, ROWS_PER_TILE)])
    pltpu.sync_copy(src_hbm.at[wid], sidx)
    pltpu.sync_copy(dst_hbm.at[wid], didx)
    plsc.subcore_barrier()

    def body(c, carry):
        pltpu.async_copy(y_hbm.at[sidx.at[c]], rows, sem).wait()
        pltpu.sync_copy(rows, acc.at[didx.at[c]], add=True)
        return carry

    lax.fori_loop(0, CPT, body, 0)

    plsc.subcore_barrier()
    pltpu.sync_copy(
        acc.at[pl.ds(sid * ROWS_PER_TILE, ROWS_PER_TILE)],
        out_hbm.at[cid, pl.ds(sid * ROWS_PER_TILE, ROWS_PER_TILE)],
    )


def kernel(x, edge_index, W1, b1, W2, b2):
    src = edge_index[0].astype(jnp.int32)
    dst = edge_index[1].astype(jnp.int32)
    pad = E_PAD - E
    srcp = jnp.concatenate([src, jnp.zeros((pad,), jnp.int32)])
    dstp = jnp.concatenate([dst, jnp.full((pad,), TRASH, jnp.int32)])
    srcp = srcp.reshape(NW, CPT, CHUNK)
    dstp = dstp.reshape(NW, CPT, CHUNK)

    zeros1 = jnp.zeros((DEG_RPT,), jnp.float32)
    zeros2 = jnp.zeros((ROWS_PER_TILE, D), jnp.float32)

    deg_parts = _deg_sc(dstp, zeros1)
    deg = deg_parts[0, :N_NODES] + deg_parts[1, :N_NODES] + 1.0
    dis = lax.rsqrt(deg)[:, None]

    y1 = (x @ W1) * dis
    agg1 = _agg_sc(y1, srcp, dstp, zeros2)
    h = dis * (agg1[0, :N_NODES] + agg1[1, :N_NODES] + y1) + b1
    h = jnp.maximum(h, 0.0)

    y2 = (h @ W2) * dis
    agg2 = _agg_sc(y2, srcp, dstp, zeros2)
    return dis * (agg2[0, :N_NODES] + agg2[1, :N_NODES] + y2) + b2


# speedup vs baseline: 1.5506x; 1.0287x over previous
import functools

import jax
import jax.numpy as jnp
from jax import lax
from jax.experimental import pallas as pl
from jax.experimental.pallas import tpu as pltpu
from jax.experimental.pallas import tpu_sc as plsc

N_NODES = 10000
D = 128
E = 320000

NC = 2
NS = 16
NW = NC * NS

CHUNK = 128
CPT = 79
EPT = CPT * CHUNK
E_PAD = NW * EPT
ROWS_PER_TILE = 640
ACC_ROWS = NS * ROWS_PER_TILE
DEG_RPT = 640
DEG_ROWS = NS * DEG_RPT
TRASH = N_NODES

_mesh = plsc.VectorSubcoreMesh(core_axis_name="c", subcore_axis_name="s")


@functools.partial(
    pl.kernel,
    out_type=jax.ShapeDtypeStruct((NC, DEG_ROWS), jnp.float32),
    mesh=_mesh,
    scratch_types=[
        pltpu.VMEM((CHUNK,), jnp.float32),
        pltpu.VMEM((CPT, CHUNK), jnp.int32),
        pltpu.VMEM_SHARED((DEG_ROWS,), jnp.float32),
    ],
)
def _deg_sc(dst_hbm, zeros_hbm, out_hbm, ones_v, didx, acc):
    cid = lax.axis_index("c")
    sid = lax.axis_index("s")
    wid = sid * NC + cid
    for j in range(CHUNK // 16):
        ones_v[pl.ds(j * 16, 16)] = jnp.ones((16,), jnp.float32)
    pltpu.sync_copy(zeros_hbm, acc.at[pl.ds(sid * DEG_RPT, DEG_RPT)])
    pltpu.sync_copy(dst_hbm.at[wid], didx)
    plsc.subcore_barrier()

    def body(c, carry):
        pltpu.sync_copy(ones_v, acc.at[didx.at[c]], add=True)
        return carry

    lax.fori_loop(0, CPT, body, 0)
    plsc.subcore_barrier()
    pltpu.sync_copy(
        acc.at[pl.ds(sid * DEG_RPT, DEG_RPT)],
        out_hbm.at[cid, pl.ds(sid * DEG_RPT, DEG_RPT)],
    )


@functools.partial(
    pl.kernel,
    out_type=jax.ShapeDtypeStruct((NC, ACC_ROWS, D), jnp.float32),
    mesh=_mesh,
    scratch_types=[
        pltpu.VMEM((CPT, CHUNK), jnp.int32),
        pltpu.VMEM((CPT, CHUNK), jnp.int32),
        pltpu.VMEM((CHUNK, D), jnp.float32),
        pltpu.SemaphoreType.DMA,
        pltpu.VMEM_SHARED((ACC_ROWS, D), jnp.float32),
    ],
)
def _agg_sc(y_hbm, src_hbm, dst_hbm, zeros_hbm, out_hbm,
            sidx, didx, rows, sem, acc):
    cid = lax.axis_index("c")
    sid = lax.axis_index("s")
    wid = sid * NC + cid
    pltpu.sync_copy(zeros_hbm, acc.at[pl.ds(sid * ROWS_PER_TILE, ROWS_PER_TILE)])
    pltpu.sync_copy(src_hbm.at[wid], sidx)
    pltpu.sync_copy(dst_hbm.at[wid], didx)
    plsc.subcore_barrier()

    def body(c, carry):
        pltpu.sync_copy(y_hbm.at[sidx.at[c]], rows)
        pltpu.sync_copy(rows, acc.at[didx.at[c]], add=True)
        return carry

    lax.fori_loop(0, CPT, body, 0)

    plsc.subcore_barrier()
    pltpu.sync_copy(
        acc.at[pl.ds(sid * ROWS_PER_TILE, ROWS_PER_TILE)],
        out_hbm.at[cid, pl.ds(sid * ROWS_PER_TILE, ROWS_PER_TILE)],
    )


def kernel(x, edge_index, W1, b1, W2, b2):
    src = edge_index[0].astype(jnp.int32)
    dst = edge_index[1].astype(jnp.int32)
    pad = E_PAD - E
    srcp = jnp.concatenate([src, jnp.zeros((pad,), jnp.int32)])
    dstp = jnp.concatenate([dst, jnp.full((pad,), TRASH, jnp.int32)])
    srcp = srcp.reshape(NW, CPT, CHUNK)
    dstp = dstp.reshape(NW, CPT, CHUNK)

    zeros1 = jnp.zeros((DEG_RPT,), jnp.float32)
    zeros2 = jnp.zeros((ROWS_PER_TILE, D), jnp.float32)

    deg_parts = _deg_sc(dstp, zeros1)
    deg = deg_parts[0, :N_NODES] + deg_parts[1, :N_NODES] + 1.0
    dis = lax.rsqrt(deg)[:, None]

    y1 = (x @ W1) * dis
    agg1 = _agg_sc(y1, srcp, dstp, zeros2)
    h = dis * (agg1[0, :N_NODES] + agg1[1, :N_NODES] + y1) + b1
    h = jnp.maximum(h, 0.0)

    y2 = (h @ W2) * dis
    agg2 = _agg_sc(y2, srcp, dstp, zeros2)
    return dis * (agg2[0, :N_NODES] + agg2[1, :N_NODES] + y2) + b2
